# baseline (speedup 1.0000x reference)
import functools

import jax
import jax.numpy as jnp
from jax import lax
from jax.experimental import pallas as pl
from jax.experimental.pallas import tpu as pltpu
from jax.experimental.pallas import tpu_sc as plsc

_INTERPRET = False

_NR = 128
_NRANGES = 96
_NP = _NR * _NRANGES
_NPC = 10240
_DP = 768
_NC = 8
_DC = _DP // _NC
_SB = 2000
_GB = 64
_EPAD = 50176
_DUMMY = _NR << 14


def _matmul_block_kernel(x_ref, w_ref, o_ref):
    o_ref[...] = jnp.dot(x_ref[...], w_ref[...], preferred_element_type=jnp.float32)


def _matmul(x, w, bm=None):
    m, k = x.shape
    n = w.shape[1]
    if bm is None or m % bm != 0:
        bm = m
    return pl.pallas_call(
        _matmul_block_kernel,
        out_shape=jax.ShapeDtypeStruct((m, n), jnp.float32),
        grid=(m // bm,),
        in_specs=[
            pl.BlockSpec((bm, k), lambda i: (i, 0)),
            pl.BlockSpec((k, n), lambda i: (0, 0)),
        ],
        out_specs=pl.BlockSpec((bm, n), lambda i: (i, 0)),
        interpret=_INTERPRET,
    )(x, w)


def _ab_kernel(x_ref, wd_ref, ws_ref, a_ref, b_ref):
    xv = x_ref[...]
    av = jnp.dot(xv, wd_ref[...], preferred_element_type=jnp.float32)
    cnt_col = lax.broadcasted_iota(jnp.int32, av.shape, 1) == (x_ref.shape[1])
    a_ref[...] = av + jnp.where(cnt_col, 1.0, 0.0)
    b_ref[...] = jnp.dot(xv, ws_ref[...], preferred_element_type=jnp.float32)


def _compute_ab(x, wd, ws, bm=1024):
    n, d = x.shape
    dout = wd.shape[1]
    return pl.pallas_call(
        _ab_kernel,
        out_shape=[jax.ShapeDtypeStruct((_NP, dout), jnp.float32)] * 2,
        grid=(n // bm,),
        in_specs=[
            pl.BlockSpec((bm, d), lambda i: (i, 0)),
            pl.BlockSpec((d, dout), lambda i: (0, 0)),
            pl.BlockSpec((d, dout), lambda i: (0, 0)),
        ],
        out_specs=[pl.BlockSpec((bm, dout), lambda i: (i, 0))] * 2,
        interpret=_INTERPRET,
    )(x, wd, ws)


def _combine_kernel(x_ref, s_ref, mn_ref, mx_ref, ssq_ref, aw_ref,
                    pl_ref, bias_ref, o_ref):
    d = x_ref.shape[1]
    cnt = s_ref[...][:, d:d + 1]
    has = cnt > 0.0
    c1 = jnp.maximum(cnt, 1.0)
    s = s_ref[...][:, :d]
    mean = jnp.where(has, s / c1, 0.0)
    msq = jnp.where(has, ssq_ref[...][:, :d] / c1, 0.0)
    var = msq - mean * mean
    std = aw_ref[0, 4] * jnp.sqrt(jnp.maximum(var, 0.0) + 1e-5)
    mn = jnp.where(has, mn_ref[...][:, :d], 0.0)
    mx = jnp.where(has, mx_ref[...][:, :d], 0.0)
    acc = jnp.dot(x_ref[...], pl_ref[0], preferred_element_type=jnp.float32)
    acc += jnp.dot(aw_ref[0, 0] * s, pl_ref[1], preferred_element_type=jnp.float32)
    acc += jnp.dot(aw_ref[0, 1] * mean, pl_ref[2], preferred_element_type=jnp.float32)
    acc += jnp.dot(aw_ref[0, 2] * mn, pl_ref[3], preferred_element_type=jnp.float32)
    acc += jnp.dot(aw_ref[0, 3] * mx, pl_ref[4], preferred_element_type=jnp.float32)
    acc += jnp.dot(std, pl_ref[5], preferred_element_type=jnp.float32)
    o_ref[...] = acc + bias_ref[...]


def _combine(x, s, mn, mx, ssq, aggw, plw, bias, bm=512):
    n, d = x.shape
    agg_spec = pl.BlockSpec((bm, _DP), lambda i: (i, 0))
    return pl.pallas_call(
        _combine_kernel,
        out_shape=jax.ShapeDtypeStruct((n, d), jnp.float32),
        grid=(n // bm,),
        in_specs=[
            pl.BlockSpec((bm, d), lambda i: (i, 0)),
            agg_spec, agg_spec, agg_spec, agg_spec,
            pl.BlockSpec((1, 8), lambda i: (0, 0)),
            pl.BlockSpec((6, d, d), lambda i: (0, 0, 0)),
            pl.BlockSpec((1, d), lambda i: (0, 0)),
        ],
        out_specs=pl.BlockSpec((bm, d), lambda i: (i, 0)),
        interpret=_INTERPRET,
    )(x, s, mn, mx, ssq, aggw, plw, bias)


def _colstats_kernel(h_ref, o_ref):
    @pl.when(pl.program_id(0) == 0)
    def _():
        o_ref[...] = jnp.zeros_like(o_ref)
    hv = h_ref[...]
    o_ref[0, :] += jnp.sum(hv, axis=0)
    o_ref[1, :] += jnp.sum(hv * hv, axis=0)


def _bn_relu_apply_kernel(h_ref, st_ref, g_ref, b_ref, o_ref):
    n_total = st_ref[2, 0]
    mu = st_ref[0, :] / n_total
    var = st_ref[1, :] / n_total - mu * mu
    inv = jax.lax.rsqrt(var + 1e-5)
    o_ref[...] = jnp.maximum(
        (h_ref[...] - mu[None, :]) * (inv * g_ref[0, :])[None, :] + b_ref[0, :][None, :],
        0.0)


def _bn_relu(h, g, b, bm=1000):
    n, d = h.shape
    stats = pl.pallas_call(
        _colstats_kernel,
        out_shape=jax.ShapeDtypeStruct((3, d), jnp.float32),
        grid=(n // bm,),
        in_specs=[pl.BlockSpec((bm, d), lambda i: (i, 0))],
        out_specs=pl.BlockSpec((3, d), lambda i: (0, 0)),
        interpret=_INTERPRET,
    )(h)
    stats = stats.at[2, 0].set(float(n))
    return pl.pallas_call(
        _bn_relu_apply_kernel,
        out_shape=jax.ShapeDtypeStruct((n, d), jnp.float32),
        grid=(n // bm,),
        in_specs=[
            pl.BlockSpec((bm, d), lambda i: (i, 0)),
            pl.BlockSpec((3, d), lambda i: (0, 0)),
            pl.BlockSpec((1, d), lambda i: (0, 0)),
            pl.BlockSpec((1, d), lambda i: (0, 0)),
        ],
        out_specs=pl.BlockSpec((bm, d), lambda i: (i, 0)),
        interpret=_INTERPRET,
    )(h, stats, g.reshape(1, d), b.reshape(1, d))


def _chunk_major_kernel(x_ref, o_ref):
    bm = x_ref.shape[0]
    o_ref[...] = x_ref[...].reshape(bm, _NC, _DC).transpose(1, 0, 2)


def _chunk_major(bp, bm=1024):
    return pl.pallas_call(
        _chunk_major_kernel,
        out_shape=jax.ShapeDtypeStruct((_NC, _NP, _DC), jnp.float32),
        grid=(_NP // bm,),
        in_specs=[pl.BlockSpec((bm, _DP), lambda j: (j, 0))],
        out_specs=pl.BlockSpec((_NC, bm, _DC), lambda j: (0, j, 0)),
        interpret=_INTERPRET,
    )(bp)


def _sc_scatter_body(a_hbm, b2_hbm, t_hbm, dst_hbm, src_hbm, ea_hbm,
                     s_out, mn_out, mx_out, ssq_out,
                     a_own, t_loc, acc_s, acc_mn, acc_mx, acc_ssq,
                     e_dst, e_src, e_ea, o_packed, idx0, rb0):
    n_tiles = dst_hbm.shape[0] // _SB
    n_vecs = _SB // 16
    wid = lax.axis_index("s") * 2 + lax.axis_index("c")
    lane = lax.iota(jnp.int32, 16)

    @pl.loop(0, _EPAD // 16)
    def _(i):
        o_packed[pl.ds(i * 16, 16)] = jnp.full((16,), _DUMMY, jnp.int32)

    def scan_range(base):
        def tile_body(tp, cur):
            pltpu.sync_copy(dst_hbm.at[pl.ds(tp * _SB, _SB)], e_dst)
            pltpu.sync_copy(src_hbm.at[pl.ds(tp * _SB, _SB)], e_src)
            pltpu.sync_copy(ea_hbm.at[pl.ds(tp * _SB, _SB)], e_ea)

            def vec_body(j, cur):
                vd = e_dst[pl.ds(j * 16, 16)]
                mask = (vd >= base) & (vd < base + _NR)
                packed = (e_src[pl.ds(j * 16, 16)]
                          | ((vd - base) << 14)
                          | (e_ea[pl.ds(j * 16, 16)] << 22))
                pos = plsc.cumsum(mask.astype(jnp.int32)) + (cur - 1)
                plsc.store_scatter(o_packed, [pos], packed, mask=mask)
                pc = plsc.all_reduce_population_count(mask)[0]
                return cur + pc

            return lax.fori_loop(0, n_vecs, vec_body, cur)

        cur = lax.fori_loop(0, n_tiles, tile_body, jnp.int32(0))
        for i in range(_GB // 16):
            plsc.store_scatter(o_packed, [cur + i * 16 + lane],
                               jnp.full((16,), _DUMMY, jnp.int32))
        return cur

    def process_chunk(c, count, base):
        col = c * _DC
        pltpu.sync_copy(a_hbm.at[pl.ds(base, _NR), pl.ds(col, _DC)],
                        a_own.at[pl.ds(0, _NR)])
        pltpu.sync_copy(t_hbm.at[:, pl.ds(col, _DC)], t_loc)

        @pl.loop(0, _NR + 8)
        def _(i):
            for g in range(_DC // 16):
                sl = pl.ds(g * 16, 16)
                acc_s[i, sl] = jnp.zeros((16,), jnp.float32)
                acc_ssq[i, sl] = jnp.zeros((16,), jnp.float32)
                acc_mn[i, sl] = jnp.full((16,), jnp.inf, jnp.float32)
                acc_mx[i, sl] = jnp.full((16,), -jnp.inf, jnp.float32)

        n_batches = (count + _GB - 1) // _GB

        def batch_body(bi, _):
            b0 = bi * _GB
            for i in range(_GB // 16):
                wv = o_packed[pl.ds(b0 + i * 16, 16)]
                idx0[pl.ds(i * 16, 16)] = (wv & 16383) + c * _NP
            pltpu.sync_copy(b2_hbm.at[idx0], rb0)

            for i in range(_GB // 16):
                wv = o_packed[pl.ds(b0 + i * 16, 16)]
                for ln in range(16):
                    w = wv[ln]
                    dl = (w >> 14) & 255
                    eav = (w >> 22) & 31
                    k = i * 16 + ln
                    for g in range(_DC // 16):
                        sl = pl.ds(g * 16, 16)
                        m = a_own[dl, sl] + rb0[k, sl] + t_loc[eav, sl]
                        acc_s[dl, sl] = acc_s[dl, sl] + m
                        acc_ssq[dl, sl] = acc_ssq[dl, sl] + m * m
                        acc_mn[dl, sl] = jnp.minimum(acc_mn[dl, sl], m)
                        acc_mx[dl, sl] = jnp.maximum(acc_mx[dl, sl], m)
            return 0

        lax.fori_loop(0, n_batches, batch_body, 0)

        pltpu.sync_copy(acc_s.at[pl.ds(0, _NR)],
                        s_out.at[pl.ds(base, _NR), pl.ds(col, _DC)])
        pltpu.sync_copy(acc_mn.at[pl.ds(0, _NR)],
                        mn_out.at[pl.ds(base, _NR), pl.ds(col, _DC)])
        pltpu.sync_copy(acc_mx.at[pl.ds(0, _NR)],
                        mx_out.at[pl.ds(base, _NR), pl.ds(col, _DC)])
        pltpu.sync_copy(acc_ssq.at[pl.ds(0, _NR)],
                        ssq_out.at[pl.ds(base, _NR), pl.ds(col, _DC)])

    def range_body(rr, _):
        base = (wid * 3 + rr) * _NR
        count = scan_range(base)

        def chunk_body(c, _):
            process_chunk(c, count, base)
            return 0

        lax.fori_loop(0, _NC, chunk_body, 0)
        return 0

    lax.fori_loop(0, 3, range_body, 0)


def _sc_scatter(a_p, b_t, t_table, dst, src, ea):
    b2 = b_t.reshape(_NC * _NP, _DC)
    f32 = jnp.float32
    mesh = plsc.VectorSubcoreMesh(core_axis_name="c", subcore_axis_name="s")
    kern = pl.kernel(
        _sc_scatter_body,
        out_type=[jax.ShapeDtypeStruct((_NP, _DP), f32)] * 4,
        mesh=mesh,
        compiler_params=pltpu.CompilerParams(use_tc_tiling_on_sc=False, needs_layout_passes=False),
        scratch_types=[
            pltpu.VMEM((_NR + 8, _DC), f32),
            pltpu.VMEM((32, _DC), f32),
            pltpu.VMEM((_NR + 8, _DC), f32),
            pltpu.VMEM((_NR + 8, _DC), f32),
            pltpu.VMEM((_NR + 8, _DC), f32),
            pltpu.VMEM((_NR + 8, _DC), f32),
            pltpu.VMEM((_SB,), jnp.int32),
            pltpu.VMEM((_SB,), jnp.int32),
            pltpu.VMEM((_SB,), jnp.int32),
            pltpu.VMEM((_EPAD,), jnp.int32),
            pltpu.VMEM((_GB,), jnp.int32),
            pltpu.VMEM((_GB, _DC), f32),
        ],
    )
    return kern(a_p, b2, t_table, dst, src, ea)


def _ttable_kernel(emb_ref, encw_ref, encb_ref, we_ref, preb_ref, o_ref):
    enc = jnp.dot(emb_ref[...], encw_ref[...], preferred_element_type=jnp.float32)
    enc = enc + encb_ref[...]
    o_ref[...] = jnp.dot(enc, we_ref[...], preferred_element_type=jnp.float32) + preb_ref[...]


def _ttable(edge_emb, enc_W, enc_b, we_p, pre_b_p):
    d = enc_W.shape[1]
    emb32 = jnp.zeros((32, 16), jnp.float32).at[:20, :10].set(edge_emb)
    encw16 = jnp.zeros((16, d), jnp.float32).at[:10].set(enc_W)
    return pl.pallas_call(
        _ttable_kernel,
        out_shape=jax.ShapeDtypeStruct((32, _DP), jnp.float32),
        interpret=_INTERPRET,
    )(emb32, encw16, enc_b.reshape(1, d), we_p, pre_b_p.reshape(1, _DP))


def _conv_layer(x_p, src, dst, edge_attr, edge_emb, aggw8,
                enc_W, enc_b, pre_W, pre_b, post_W, post_b, lin_W, lin_b):
    d = x_p.shape[1]
    pad_c = ((0, 0), (0, _DP - d))
    wd = jnp.pad(pre_W[:d], pad_c)
    ws = jnp.pad(pre_W[d:2 * d], pad_c)
    we = jnp.pad(pre_W[2 * d:], pad_c)
    pre_b_p = jnp.pad(pre_b, (0, _DP - d))
    a_p, b_p = _compute_ab(x_p, wd, ws)
    b_t = _chunk_major(b_p)
    t_table = _ttable(edge_emb, enc_W, enc_b, we, pre_b_p)
    s, mn, mx, ssq = _sc_scatter(a_p, b_t, t_table, dst, src, edge_attr)
    pw = jnp.concatenate([post_W, post_b[None, :]], axis=0)
    plw_full = _matmul(pw, lin_W, bm=None)
    bias = (plw_full[6 * d] + lin_b)[None, :]
    plw = plw_full[:6 * d].reshape(6, d, d)
    return _combine(x_p, s, mn, mx, ssq, aggw8, plw, bias)


def kernel(x, edge_index, edge_attr, edge_emb, agg_weights,
           enc_W0, enc_b0, pre_W0, pre_b0, post_W0, post_b0, lin_W0, lin_b0, bn_g0, bn_b0,
           enc_W1, enc_b1, pre_W1, pre_b1, post_W1, post_b1, lin_W1, lin_b1, bn_g1, bn_b1):
    n, d = x.shape
    src = edge_index[0]
    dst = edge_index[1]
    aggw8 = jnp.zeros((1, 8), jnp.float32).at[0, :5].set(agg_weights)

    h = x
    for enc_W, enc_b, pre_W, pre_b, post_W, post_b, lin_W, lin_b, bn_g, bn_b in (
        (enc_W0, enc_b0, pre_W0, pre_b0, post_W0, post_b0, lin_W0, lin_b0, bn_g0, bn_b0),
        (enc_W1, enc_b1, pre_W1, pre_b1, post_W1, post_b1, lin_W1, lin_b1, bn_g1, bn_b1),
    ):
        h_p = jnp.pad(h, ((0, _NPC - n), (0, 0)))
        h = _conv_layer(h_p, src, dst, edge_attr, edge_emb, aggw8,
                        enc_W, enc_b, pre_W, pre_b, post_W, post_b, lin_W, lin_b)[:n]
        h = _bn_relu(h, bn_g, bn_b)
    return h


# speedup vs baseline: 1.4429x; 1.4429x over previous
import functools

import jax
import jax.numpy as jnp
from jax import lax
from jax.experimental import pallas as pl
from jax.experimental.pallas import tpu as pltpu
from jax.experimental.pallas import tpu_sc as plsc

_INTERPRET = False

_NR = 128
_NRANGES = 96
_NP = _NR * _NRANGES
_NPC = 10240
_DP = 768
_NC = 8
_DC = _DP // _NC
_SB = 2000
_GB = 16
_EPAD = 50176
_DUMMY = _NR << 14


def _matmul_block_kernel(x_ref, w_ref, o_ref):
    o_ref[...] = jnp.dot(x_ref[...], w_ref[...], preferred_element_type=jnp.float32)


def _matmul(x, w, bm=None):
    m, k = x.shape
    n = w.shape[1]
    if bm is None or m % bm != 0:
        bm = m
    return pl.pallas_call(
        _matmul_block_kernel,
        out_shape=jax.ShapeDtypeStruct((m, n), jnp.float32),
        grid=(m // bm,),
        in_specs=[
            pl.BlockSpec((bm, k), lambda i: (i, 0)),
            pl.BlockSpec((k, n), lambda i: (0, 0)),
        ],
        out_specs=pl.BlockSpec((bm, n), lambda i: (i, 0)),
        interpret=_INTERPRET,
    )(x, w)


def _ab_kernel(x_ref, wd_ref, ws_ref, a_ref, b_ref):
    xv = x_ref[...]
    av = jnp.dot(xv, wd_ref[...], preferred_element_type=jnp.float32)
    cnt_col = lax.broadcasted_iota(jnp.int32, av.shape, 1) == (x_ref.shape[1])
    a_ref[...] = av + jnp.where(cnt_col, 1.0, 0.0)
    b_ref[...] = jnp.dot(xv, ws_ref[...], preferred_element_type=jnp.float32)


def _compute_ab(x, wd, ws, bm=1024):
    n, d = x.shape
    dout = wd.shape[1]
    return pl.pallas_call(
        _ab_kernel,
        out_shape=[jax.ShapeDtypeStruct((_NP, dout), jnp.float32)] * 2,
        grid=(n // bm,),
        in_specs=[
            pl.BlockSpec((bm, d), lambda i: (i, 0)),
            pl.BlockSpec((d, dout), lambda i: (0, 0)),
            pl.BlockSpec((d, dout), lambda i: (0, 0)),
        ],
        out_specs=[pl.BlockSpec((bm, dout), lambda i: (i, 0))] * 2,
        interpret=_INTERPRET,
    )(x, wd, ws)


def _combine_kernel(x_ref, s_ref, mn_ref, mx_ref, ssq_ref, aw_ref,
                    pl_ref, bias_ref, o_ref):
    d = x_ref.shape[1]
    cnt = s_ref[...][:, d:d + 1]
    has = cnt > 0.0
    c1 = jnp.maximum(cnt, 1.0)
    s = s_ref[...][:, :d]
    mean = jnp.where(has, s / c1, 0.0)
    msq = jnp.where(has, ssq_ref[...][:, :d] / c1, 0.0)
    var = msq - mean * mean
    std = aw_ref[0, 4] * jnp.sqrt(jnp.maximum(var, 0.0) + 1e-5)
    mn = jnp.where(has, mn_ref[...][:, :d], 0.0)
    mx = jnp.where(has, mx_ref[...][:, :d], 0.0)
    acc = jnp.dot(x_ref[...], pl_ref[0], preferred_element_type=jnp.float32)
    acc += jnp.dot(aw_ref[0, 0] * s, pl_ref[1], preferred_element_type=jnp.float32)
    acc += jnp.dot(aw_ref[0, 1] * mean, pl_ref[2], preferred_element_type=jnp.float32)
    acc += jnp.dot(aw_ref[0, 2] * mn, pl_ref[3], preferred_element_type=jnp.float32)
    acc += jnp.dot(aw_ref[0, 3] * mx, pl_ref[4], preferred_element_type=jnp.float32)
    acc += jnp.dot(std, pl_ref[5], preferred_element_type=jnp.float32)
    o_ref[...] = acc + bias_ref[...]


def _combine(x, s, mn, mx, ssq, aggw, plw, bias, bm=512):
    n, d = x.shape
    agg_spec = pl.BlockSpec((bm, _DP), lambda i: (i, 0))
    return pl.pallas_call(
        _combine_kernel,
        out_shape=jax.ShapeDtypeStruct((n, d), jnp.float32),
        grid=(n // bm,),
        in_specs=[
            pl.BlockSpec((bm, d), lambda i: (i, 0)),
            agg_spec, agg_spec, agg_spec, agg_spec,
            pl.BlockSpec((1, 8), lambda i: (0, 0)),
            pl.BlockSpec((6, d, d), lambda i: (0, 0, 0)),
            pl.BlockSpec((1, d), lambda i: (0, 0)),
        ],
        out_specs=pl.BlockSpec((bm, d), lambda i: (i, 0)),
        interpret=_INTERPRET,
    )(x, s, mn, mx, ssq, aggw, plw, bias)


def _colstats_kernel(h_ref, o_ref):
    @pl.when(pl.program_id(0) == 0)
    def _():
        o_ref[...] = jnp.zeros_like(o_ref)
    hv = h_ref[...]
    o_ref[0, :] += jnp.sum(hv, axis=0)
    o_ref[1, :] += jnp.sum(hv * hv, axis=0)


def _bn_relu_apply_kernel(h_ref, st_ref, g_ref, b_ref, o_ref):
    n_total = st_ref[2, 0]
    mu = st_ref[0, :] / n_total
    var = st_ref[1, :] / n_total - mu * mu
    inv = jax.lax.rsqrt(var + 1e-5)
    o_ref[...] = jnp.maximum(
        (h_ref[...] - mu[None, :]) * (inv * g_ref[0, :])[None, :] + b_ref[0, :][None, :],
        0.0)


def _bn_relu(h, g, b, bm=1000):
    n, d = h.shape
    stats = pl.pallas_call(
        _colstats_kernel,
        out_shape=jax.ShapeDtypeStruct((3, d), jnp.float32),
        grid=(n // bm,),
        in_specs=[pl.BlockSpec((bm, d), lambda i: (i, 0))],
        out_specs=pl.BlockSpec((3, d), lambda i: (0, 0)),
        interpret=_INTERPRET,
    )(h)
    stats = stats.at[2, 0].set(float(n))
    return pl.pallas_call(
        _bn_relu_apply_kernel,
        out_shape=jax.ShapeDtypeStruct((n, d), jnp.float32),
        grid=(n // bm,),
        in_specs=[
            pl.BlockSpec((bm, d), lambda i: (i, 0)),
            pl.BlockSpec((3, d), lambda i: (0, 0)),
            pl.BlockSpec((1, d), lambda i: (0, 0)),
            pl.BlockSpec((1, d), lambda i: (0, 0)),
        ],
        out_specs=pl.BlockSpec((bm, d), lambda i: (i, 0)),
        interpret=_INTERPRET,
    )(h, stats, g.reshape(1, d), b.reshape(1, d))


def _chunk_major_kernel(x_ref, o_ref):
    bm = x_ref.shape[0]
    o_ref[...] = x_ref[...].reshape(bm, _NC, _DC).transpose(1, 0, 2)


def _chunk_major(bp, bm=1024):
    return pl.pallas_call(
        _chunk_major_kernel,
        out_shape=jax.ShapeDtypeStruct((_NC, _NP, _DC), jnp.float32),
        grid=(_NP // bm,),
        in_specs=[pl.BlockSpec((bm, _DP), lambda j: (j, 0))],
        out_specs=pl.BlockSpec((_NC, bm, _DC), lambda j: (0, j, 0)),
        interpret=_INTERPRET,
    )(bp)


def _sc_scatter_body(a_hbm, b2_hbm, t_hbm, dst_hbm, src_hbm, ea_hbm,
                     s_out, mn_out, mx_out, ssq_out,
                     a_own, t_loc, acc_s, acc_mn, acc_mx, acc_ssq,
                     e_dst, e_src, e_ea, o_packed, idx0, rb0):
    n_tiles = dst_hbm.shape[0] // _SB
    n_vecs = _SB // 16
    wid = lax.axis_index("s") * 2 + lax.axis_index("c")
    lane = lax.iota(jnp.int32, 16)

    @pl.loop(0, _EPAD // 16)
    def _(i):
        o_packed[pl.ds(i * 16, 16)] = jnp.full((16,), _DUMMY, jnp.int32)

    def scan_range(base):
        def tile_body(tp, cur):
            pltpu.sync_copy(dst_hbm.at[pl.ds(tp * _SB, _SB)], e_dst)
            pltpu.sync_copy(src_hbm.at[pl.ds(tp * _SB, _SB)], e_src)
            pltpu.sync_copy(ea_hbm.at[pl.ds(tp * _SB, _SB)], e_ea)

            def vec_body(j, cur):
                vd = e_dst[pl.ds(j * 16, 16)]
                mask = (vd >= base) & (vd < base + _NR)
                packed = (e_src[pl.ds(j * 16, 16)]
                          | ((vd - base) << 14)
                          | (e_ea[pl.ds(j * 16, 16)] << 22))
                pos = plsc.cumsum(mask.astype(jnp.int32)) + (cur - 1)
                plsc.store_scatter(o_packed, [pos], packed, mask=mask)
                pc = plsc.all_reduce_population_count(mask)[0]
                return cur + pc

            return lax.fori_loop(0, n_vecs, vec_body, cur)

        cur = lax.fori_loop(0, n_tiles, tile_body, jnp.int32(0))
        for i in range(_GB // 16):
            plsc.store_scatter(o_packed, [cur + i * 16 + lane],
                               jnp.full((16,), _DUMMY, jnp.int32))
        return cur

    def process_chunk(c, count, base):
        col = c * _DC
        pltpu.sync_copy(a_hbm.at[pl.ds(base, _NR), pl.ds(col, _DC)],
                        a_own.at[pl.ds(0, _NR)])
        pltpu.sync_copy(t_hbm.at[:, pl.ds(col, _DC)], t_loc)

        @pl.loop(0, _NR + 8)
        def _(i):
            for g in range(_DC // 16):
                sl = pl.ds(g * 16, 16)
                acc_s[i, sl] = jnp.zeros((16,), jnp.float32)
                acc_ssq[i, sl] = jnp.zeros((16,), jnp.float32)
                acc_mn[i, sl] = jnp.full((16,), jnp.inf, jnp.float32)
                acc_mx[i, sl] = jnp.full((16,), -jnp.inf, jnp.float32)

        n_batches = (count + _GB - 1) // _GB

        def batch_body(bi, _):
            b0 = bi * _GB
            for i in range(_GB // 16):
                wv = o_packed[pl.ds(b0 + i * 16, 16)]
                idx0[pl.ds(i * 16, 16)] = (wv & 16383) + c * _NP
            pltpu.sync_copy(b2_hbm.at[idx0], rb0)

            for i in range(_GB // 16):
                wv = o_packed[pl.ds(b0 + i * 16, 16)]
                for ln in range(16):
                    w = wv[ln]
                    dl = (w >> 14) & 255
                    eav = (w >> 22) & 31
                    k = i * 16 + ln
                    for g in range(_DC // 16):
                        sl = pl.ds(g * 16, 16)
                        m = a_own[dl, sl] + rb0[k, sl] + t_loc[eav, sl]
                        acc_s[dl, sl] = acc_s[dl, sl] + m
                        acc_ssq[dl, sl] = acc_ssq[dl, sl] + m * m
                        acc_mn[dl, sl] = jnp.minimum(acc_mn[dl, sl], m)
                        acc_mx[dl, sl] = jnp.maximum(acc_mx[dl, sl], m)
            return 0

        lax.fori_loop(0, n_batches, batch_body, 0)

        pltpu.sync_copy(acc_s.at[pl.ds(0, _NR)],
                        s_out.at[pl.ds(base, _NR), pl.ds(col, _DC)])
        pltpu.sync_copy(acc_mn.at[pl.ds(0, _NR)],
                        mn_out.at[pl.ds(base, _NR), pl.ds(col, _DC)])
        pltpu.sync_copy(acc_mx.at[pl.ds(0, _NR)],
                        mx_out.at[pl.ds(base, _NR), pl.ds(col, _DC)])
        pltpu.sync_copy(acc_ssq.at[pl.ds(0, _NR)],
                        ssq_out.at[pl.ds(base, _NR), pl.ds(col, _DC)])

    def range_body(rr, _):
        base = (wid * 3 + rr) * _NR
        with jax.named_scope("edge_scan"):
            count = scan_range(base)

        def chunk_body(c, _):
            process_chunk(c, count, base)
            return 0

        with jax.named_scope("chunk_accum"):
            lax.fori_loop(0, _NC, chunk_body, 0)
        return 0

    lax.fori_loop(0, 3, range_body, 0)


def _sc_scatter(a_p, b_t, t_table, dst, src, ea):
    b2 = b_t.reshape(_NC * _NP, _DC)
    f32 = jnp.float32
    mesh = plsc.VectorSubcoreMesh(core_axis_name="c", subcore_axis_name="s")
    kern = pl.kernel(
        _sc_scatter_body,
        out_type=[jax.ShapeDtypeStruct((_NP, _DP), f32)] * 4,
        mesh=mesh,
        compiler_params=pltpu.CompilerParams(use_tc_tiling_on_sc=False, needs_layout_passes=False),
        scratch_types=[
            pltpu.VMEM((_NR + 8, _DC), f32),
            pltpu.VMEM((32, _DC), f32),
            pltpu.VMEM((_NR + 8, _DC), f32),
            pltpu.VMEM((_NR + 8, _DC), f32),
            pltpu.VMEM((_NR + 8, _DC), f32),
            pltpu.VMEM((_NR + 8, _DC), f32),
            pltpu.VMEM((_SB,), jnp.int32),
            pltpu.VMEM((_SB,), jnp.int32),
            pltpu.VMEM((_SB,), jnp.int32),
            pltpu.VMEM((_EPAD,), jnp.int32),
            pltpu.VMEM((_GB,), jnp.int32),
            pltpu.VMEM((_GB, _DC), f32),
        ],
    )
    return kern(a_p, b2, t_table, dst, src, ea)


def _ttable_kernel(emb_ref, encw_ref, encb_ref, we_ref, preb_ref, o_ref):
    enc = jnp.dot(emb_ref[...], encw_ref[...], preferred_element_type=jnp.float32)
    enc = enc + encb_ref[...]
    o_ref[...] = jnp.dot(enc, we_ref[...], preferred_element_type=jnp.float32) + preb_ref[...]


def _ttable(edge_emb, enc_W, enc_b, we_p, pre_b_p):
    d = enc_W.shape[1]
    emb32 = jnp.zeros((32, 16), jnp.float32).at[:20, :10].set(edge_emb)
    encw16 = jnp.zeros((16, d), jnp.float32).at[:10].set(enc_W)
    return pl.pallas_call(
        _ttable_kernel,
        out_shape=jax.ShapeDtypeStruct((32, _DP), jnp.float32),
        interpret=_INTERPRET,
    )(emb32, encw16, enc_b.reshape(1, d), we_p, pre_b_p.reshape(1, _DP))


def _conv_layer(x_p, src, dst, edge_attr, edge_emb, aggw8,
                enc_W, enc_b, pre_W, pre_b, post_W, post_b, lin_W, lin_b):
    d = x_p.shape[1]
    pad_c = ((0, 0), (0, _DP - d))
    wd = jnp.pad(pre_W[:d], pad_c)
    ws = jnp.pad(pre_W[d:2 * d], pad_c)
    we = jnp.pad(pre_W[2 * d:], pad_c)
    pre_b_p = jnp.pad(pre_b, (0, _DP - d))
    a_p, b_p = _compute_ab(x_p, wd, ws)
    b_t = _chunk_major(b_p)
    t_table = _ttable(edge_emb, enc_W, enc_b, we, pre_b_p)
    s, mn, mx, ssq = _sc_scatter(a_p, b_t, t_table, dst, src, edge_attr)
    pw = jnp.concatenate([post_W, post_b[None, :]], axis=0)
    plw_full = _matmul(pw, lin_W, bm=None)
    bias = (plw_full[6 * d] + lin_b)[None, :]
    plw = plw_full[:6 * d].reshape(6, d, d)
    return _combine(x_p, s, mn, mx, ssq, aggw8, plw, bias)


def kernel(x, edge_index, edge_attr, edge_emb, agg_weights,
           enc_W0, enc_b0, pre_W0, pre_b0, post_W0, post_b0, lin_W0, lin_b0, bn_g0, bn_b0,
           enc_W1, enc_b1, pre_W1, pre_b1, post_W1, post_b1, lin_W1, lin_b1, bn_g1, bn_b1):
    n, d = x.shape
    src = edge_index[0]
    dst = edge_index[1]
    aggw8 = jnp.zeros((1, 8), jnp.float32).at[0, :5].set(agg_weights)

    h = x
    for enc_W, enc_b, pre_W, pre_b, post_W, post_b, lin_W, lin_b, bn_g, bn_b in (
        (enc_W0, enc_b0, pre_W0, pre_b0, post_W0, post_b0, lin_W0, lin_b0, bn_g0, bn_b0),
        (enc_W1, enc_b1, pre_W1, pre_b1, post_W1, post_b1, lin_W1, lin_b1, bn_g1, bn_b1),
    ):
        h_p = jnp.pad(h, ((0, _NPC - n), (0, 0)))
        h = _conv_layer(h_p, src, dst, edge_attr, edge_emb, aggw8,
                        enc_W, enc_b, pre_W, pre_b, post_W, post_b, lin_W, lin_b)[:n]
        h = _bn_relu(h, bn_g, bn_b)
    return h


# speedup vs baseline: 1.5403x; 1.0675x over previous
import functools

import jax
import jax.numpy as jnp
from jax import lax
from jax.experimental import pallas as pl
from jax.experimental.pallas import tpu as pltpu
from jax.experimental.pallas import tpu_sc as plsc

_INTERPRET = False

_NR = 128
_NRANGES = 96
_NP = _NR * _NRANGES
_NPC = 10240
_DP = 768
_NC = 8
_DC = _DP // _NC
_SB = 2000
_GB = 16
_EPAD = 50176
_DUMMY = _NR << 14


def _matmul_block_kernel(x_ref, w_ref, o_ref):
    o_ref[...] = jnp.dot(x_ref[...], w_ref[...], preferred_element_type=jnp.float32)


def _matmul(x, w, bm=None):
    m, k = x.shape
    n = w.shape[1]
    if bm is None or m % bm != 0:
        bm = m
    return pl.pallas_call(
        _matmul_block_kernel,
        out_shape=jax.ShapeDtypeStruct((m, n), jnp.float32),
        grid=(m // bm,),
        in_specs=[
            pl.BlockSpec((bm, k), lambda i: (i, 0)),
            pl.BlockSpec((k, n), lambda i: (0, 0)),
        ],
        out_specs=pl.BlockSpec((bm, n), lambda i: (i, 0)),
        interpret=_INTERPRET,
    )(x, w)


def _b_kernel(x_ref, ws_ref, b_ref):
    b_ref[...] = jnp.dot(x_ref[...], ws_ref[...], preferred_element_type=jnp.float32)


def _compute_b(x, ws, bm=1024):
    n, d = x.shape
    dout = ws.shape[1]
    return pl.pallas_call(
        _b_kernel,
        out_shape=jax.ShapeDtypeStruct((_NP, dout), jnp.float32),
        grid=(n // bm,),
        in_specs=[
            pl.BlockSpec((bm, d), lambda i: (i, 0)),
            pl.BlockSpec((d, dout), lambda i: (0, 0)),
        ],
        out_specs=pl.BlockSpec((bm, dout), lambda i: (i, 0)),
        interpret=_INTERPRET,
    )(x, ws)


def _combine_kernel(x_ref, s_ref, mn_ref, mx_ref, ssq_ref, aw_ref,
                    pl_ref, m1_ref, m234_ref, bias_ref, o_ref):
    d = x_ref.shape[1]
    cnt = s_ref[...][:, d:d + 1]
    has = cnt > 0.0
    c1 = jnp.maximum(cnt, 1.0)
    su = s_ref[...][:, :d]
    mean_u = jnp.where(has, su / c1, 0.0)
    msq_u = jnp.where(has, ssq_ref[...][:, :d] / c1, 0.0)
    var = msq_u - mean_u * mean_u
    std = aw_ref[0, 4] * jnp.sqrt(jnp.maximum(var, 0.0) + 1e-5)
    mn = jnp.where(has, mn_ref[...][:, :d], 0.0)
    mx = jnp.where(has, mx_ref[...][:, :d], 0.0)
    xv = x_ref[...]
    acc = jnp.dot(xv, pl_ref[0], preferred_element_type=jnp.float32)
    acc += jnp.dot(aw_ref[0, 0] * su, pl_ref[1], preferred_element_type=jnp.float32)
    acc += jnp.dot(aw_ref[0, 1] * mean_u, pl_ref[2], preferred_element_type=jnp.float32)
    acc += jnp.dot(aw_ref[0, 2] * mn, pl_ref[3], preferred_element_type=jnp.float32)
    acc += jnp.dot(aw_ref[0, 3] * mx, pl_ref[4], preferred_element_type=jnp.float32)
    acc += jnp.dot(std, pl_ref[5], preferred_element_type=jnp.float32)
    acc += (aw_ref[0, 0] * cnt) * jnp.dot(xv, m1_ref[...], preferred_element_type=jnp.float32)
    acc += jnp.where(has, 1.0, 0.0) * jnp.dot(xv, m234_ref[...], preferred_element_type=jnp.float32)
    o_ref[...] = acc + bias_ref[...]


def _combine(x, s, mn, mx, ssq, aggw, plw, m1, m234, bias, bm=512):
    n, d = x.shape
    agg_spec = pl.BlockSpec((bm, _DP), lambda i: (i, 0))
    w_spec = pl.BlockSpec((d, d), lambda i: (0, 0))
    return pl.pallas_call(
        _combine_kernel,
        out_shape=jax.ShapeDtypeStruct((n, d), jnp.float32),
        grid=(n // bm,),
        in_specs=[
            pl.BlockSpec((bm, d), lambda i: (i, 0)),
            agg_spec, agg_spec, agg_spec, agg_spec,
            pl.BlockSpec((1, 8), lambda i: (0, 0)),
            pl.BlockSpec((6, d, d), lambda i: (0, 0, 0)),
            w_spec, w_spec,
            pl.BlockSpec((1, d), lambda i: (0, 0)),
        ],
        out_specs=pl.BlockSpec((bm, d), lambda i: (i, 0)),
        interpret=_INTERPRET,
    )(x, s, mn, mx, ssq, aggw, plw, m1, m234, bias)


def _colstats_kernel(h_ref, o_ref):
    @pl.when(pl.program_id(0) == 0)
    def _():
        o_ref[...] = jnp.zeros_like(o_ref)
    hv = h_ref[...]
    o_ref[0, :] += jnp.sum(hv, axis=0)
    o_ref[1, :] += jnp.sum(hv * hv, axis=0)


def _bn_relu_apply_kernel(h_ref, st_ref, g_ref, b_ref, o_ref):
    n_total = st_ref[2, 0]
    mu = st_ref[0, :] / n_total
    var = st_ref[1, :] / n_total - mu * mu
    inv = jax.lax.rsqrt(var + 1e-5)
    o_ref[...] = jnp.maximum(
        (h_ref[...] - mu[None, :]) * (inv * g_ref[0, :])[None, :] + b_ref[0, :][None, :],
        0.0)


def _bn_relu(h, g, b, bm=1000):
    n, d = h.shape
    stats = pl.pallas_call(
        _colstats_kernel,
        out_shape=jax.ShapeDtypeStruct((3, d), jnp.float32),
        grid=(n // bm,),
        in_specs=[pl.BlockSpec((bm, d), lambda i: (i, 0))],
        out_specs=pl.BlockSpec((3, d), lambda i: (0, 0)),
        interpret=_INTERPRET,
    )(h)
    stats = stats.at[2, 0].set(float(n))
    return pl.pallas_call(
        _bn_relu_apply_kernel,
        out_shape=jax.ShapeDtypeStruct((n, d), jnp.float32),
        grid=(n // bm,),
        in_specs=[
            pl.BlockSpec((bm, d), lambda i: (i, 0)),
            pl.BlockSpec((3, d), lambda i: (0, 0)),
            pl.BlockSpec((1, d), lambda i: (0, 0)),
            pl.BlockSpec((1, d), lambda i: (0, 0)),
        ],
        out_specs=pl.BlockSpec((bm, d), lambda i: (i, 0)),
        interpret=_INTERPRET,
    )(h, stats, g.reshape(1, d), b.reshape(1, d))


def _chunk_major_kernel(x_ref, o_ref):
    bm = x_ref.shape[0]
    o_ref[...] = x_ref[...].reshape(bm, _NC, _DC).transpose(1, 0, 2)


def _chunk_major(bp, bm=1024):
    return pl.pallas_call(
        _chunk_major_kernel,
        out_shape=jax.ShapeDtypeStruct((_NC, _NP, _DC), jnp.float32),
        grid=(_NP // bm,),
        in_specs=[pl.BlockSpec((bm, _DP), lambda j: (j, 0))],
        out_specs=pl.BlockSpec((_NC, bm, _DC), lambda j: (0, j, 0)),
        interpret=_INTERPRET,
    )(bp)


def _sc_scatter_body(b2_hbm, t_hbm, dst_hbm, src_hbm, ea_hbm,
                     s_out, mn_out, mx_out, ssq_out,
                     t_loc, acc_s, acc_mn, acc_mx, acc_ssq,
                     e_dst, e_src, e_ea, o_packed, idx0, rb0):
    n_tiles = dst_hbm.shape[0] // _SB
    n_vecs = _SB // 16
    wid = lax.axis_index("s") * 2 + lax.axis_index("c")
    lane = lax.iota(jnp.int32, 16)

    @pl.loop(0, _EPAD // 16)
    def _(i):
        o_packed[pl.ds(i * 16, 16)] = jnp.full((16,), _DUMMY, jnp.int32)

    def scan_range(base):
        def tile_body(tp, cur):
            pltpu.sync_copy(dst_hbm.at[pl.ds(tp * _SB, _SB)], e_dst)
            pltpu.sync_copy(src_hbm.at[pl.ds(tp * _SB, _SB)], e_src)
            pltpu.sync_copy(ea_hbm.at[pl.ds(tp * _SB, _SB)], e_ea)

            def vec_body(j, cur):
                vd = e_dst[pl.ds(j * 16, 16)]
                mask = (vd >= base) & (vd < base + _NR)
                packed = (e_src[pl.ds(j * 16, 16)]
                          | ((vd - base) << 14)
                          | (e_ea[pl.ds(j * 16, 16)] << 22))
                pos = plsc.cumsum(mask.astype(jnp.int32)) + (cur - 1)
                plsc.store_scatter(o_packed, [pos], packed, mask=mask)
                pc = plsc.all_reduce_population_count(mask)[0]
                return cur + pc

            return lax.fori_loop(0, n_vecs, vec_body, cur)

        cur = lax.fori_loop(0, n_tiles, tile_body, jnp.int32(0))
        for i in range(_GB // 16):
            plsc.store_scatter(o_packed, [cur + i * 16 + lane],
                               jnp.full((16,), _DUMMY, jnp.int32))
        return cur

    def process_chunk(c, count, base):
        col = c * _DC
        pltpu.sync_copy(t_hbm.at[:, pl.ds(col, _DC)], t_loc)

        @pl.loop(0, _NR + 8)
        def _(i):
            for g in range(_DC // 16):
                sl = pl.ds(g * 16, 16)
                acc_s[i, sl] = jnp.zeros((16,), jnp.float32)
                acc_ssq[i, sl] = jnp.zeros((16,), jnp.float32)
                acc_mn[i, sl] = jnp.full((16,), jnp.inf, jnp.float32)
                acc_mx[i, sl] = jnp.full((16,), -jnp.inf, jnp.float32)

        n_batches = (count + _GB - 1) // _GB

        def batch_body(bi, _):
            b0 = bi * _GB
            for i in range(_GB // 16):
                wv = o_packed[pl.ds(b0 + i * 16, 16)]
                idx0[pl.ds(i * 16, 16)] = (wv & 16383) + c * _NP
            pltpu.sync_copy(b2_hbm.at[idx0], rb0)

            for i in range(_GB // 16):
                wv = o_packed[pl.ds(b0 + i * 16, 16)]
                for ln in range(16):
                    w = wv[ln]
                    dl = (w >> 14) & 255
                    eav = (w >> 22) & 31
                    k = i * 16 + ln
                    for g in range(_DC // 16):
                        sl = pl.ds(g * 16, 16)
                        u = rb0[k, sl] + t_loc[eav, sl]
                        plsc.addupdate(acc_s.at[dl, sl], u)
                        plsc.addupdate(acc_ssq.at[dl, sl], u * u)
                        acc_mn[dl, sl] = jnp.minimum(acc_mn[dl, sl], u)
                        acc_mx[dl, sl] = jnp.maximum(acc_mx[dl, sl], u)
            return 0

        lax.fori_loop(0, n_batches, batch_body, 0)

        pltpu.sync_copy(acc_s.at[pl.ds(0, _NR)],
                        s_out.at[pl.ds(base, _NR), pl.ds(col, _DC)])
        pltpu.sync_copy(acc_mn.at[pl.ds(0, _NR)],
                        mn_out.at[pl.ds(base, _NR), pl.ds(col, _DC)])
        pltpu.sync_copy(acc_mx.at[pl.ds(0, _NR)],
                        mx_out.at[pl.ds(base, _NR), pl.ds(col, _DC)])
        pltpu.sync_copy(acc_ssq.at[pl.ds(0, _NR)],
                        ssq_out.at[pl.ds(base, _NR), pl.ds(col, _DC)])

    def range_body(rr, _):
        base = (wid * 3 + rr) * _NR
        with jax.named_scope("edge_scan"):
            count = scan_range(base)

        def chunk_body(c, _):
            process_chunk(c, count, base)
            return 0

        with jax.named_scope("chunk_accum"):
            lax.fori_loop(0, _NC, chunk_body, 0)
        return 0

    lax.fori_loop(0, 3, range_body, 0)


def _sc_scatter(b_t, t_table, dst, src, ea):
    b2 = b_t.reshape(_NC * _NP, _DC)
    f32 = jnp.float32
    mesh = plsc.VectorSubcoreMesh(core_axis_name="c", subcore_axis_name="s")
    kern = pl.kernel(
        _sc_scatter_body,
        out_type=[jax.ShapeDtypeStruct((_NP, _DP), f32)] * 4,
        mesh=mesh,
        compiler_params=pltpu.CompilerParams(use_tc_tiling_on_sc=False, needs_layout_passes=False),
        scratch_types=[
            pltpu.VMEM((32, _DC), f32),
            pltpu.VMEM((_NR + 8, _DC), f32),
            pltpu.VMEM((_NR + 8, _DC), f32),
            pltpu.VMEM((_NR + 8, _DC), f32),
            pltpu.VMEM((_NR + 8, _DC), f32),
            pltpu.VMEM((_SB,), jnp.int32),
            pltpu.VMEM((_SB,), jnp.int32),
            pltpu.VMEM((_SB,), jnp.int32),
            pltpu.VMEM((_EPAD,), jnp.int32),
            pltpu.VMEM((_GB,), jnp.int32),
            pltpu.VMEM((_GB, _DC), f32),
        ],
    )
    return kern(b2, t_table, dst, src, ea)


def _ttable_kernel(emb_ref, encw_ref, encb_ref, we_ref, preb_ref, o_ref):
    enc = jnp.dot(emb_ref[...], encw_ref[...], preferred_element_type=jnp.float32)
    enc = enc + encb_ref[...]
    out = jnp.dot(enc, we_ref[...], preferred_element_type=jnp.float32) + preb_ref[...]
    mark = lax.broadcasted_iota(jnp.int32, out.shape, 1) == we_ref.shape[0]
    o_ref[...] = out + jnp.where(mark, 1.0, 0.0)


def _ttable(edge_emb, enc_W, enc_b, we_p, pre_b_p):
    d = enc_W.shape[1]
    emb32 = jnp.zeros((32, 16), jnp.float32).at[:20, :10].set(edge_emb)
    encw16 = jnp.zeros((16, d), jnp.float32).at[:10].set(enc_W)
    return pl.pallas_call(
        _ttable_kernel,
        out_shape=jax.ShapeDtypeStruct((32, _DP), jnp.float32),
        interpret=_INTERPRET,
    )(emb32, encw16, enc_b.reshape(1, d), we_p, pre_b_p.reshape(1, _DP))


def _conv_layer(x_p, src, dst, edge_attr, edge_emb, aggw8,
                enc_W, enc_b, pre_W, pre_b, post_W, post_b, lin_W, lin_b):
    d = x_p.shape[1]
    pad_c = ((0, 0), (0, _DP - d))
    ws = jnp.pad(pre_W[d:2 * d], pad_c)
    we = jnp.pad(pre_W[2 * d:], pad_c)
    pre_b_p = jnp.pad(pre_b, (0, _DP - d))
    b_p = _compute_b(x_p, ws)
    b_t = _chunk_major(b_p)
    t_table = _ttable(edge_emb, enc_W, enc_b, we, pre_b_p)
    s, mn, mx, ssq = _sc_scatter(b_t, t_table, dst, src, edge_attr)
    pw = jnp.concatenate([post_W, post_b[None, :]], axis=0)
    plw_full = _matmul(pw, lin_W, bm=None)
    bias = (plw_full[6 * d] + lin_b)[None, :]
    plw = plw_full[:6 * d].reshape(6, d, d)
    aw = aggw8[0]
    m1 = _matmul(pre_W[:d], plw[1], bm=None)
    m234 = _matmul(pre_W[:d],
                   aw[1] * plw[2] + aw[2] * plw[3] + aw[3] * plw[4], bm=None)
    return _combine(x_p, s, mn, mx, ssq, aggw8, plw, m1, m234, bias)


def kernel(x, edge_index, edge_attr, edge_emb, agg_weights,
           enc_W0, enc_b0, pre_W0, pre_b0, post_W0, post_b0, lin_W0, lin_b0, bn_g0, bn_b0,
           enc_W1, enc_b1, pre_W1, pre_b1, post_W1, post_b1, lin_W1, lin_b1, bn_g1, bn_b1):
    n, d = x.shape
    src = edge_index[0]
    dst = edge_index[1]
    aggw8 = jnp.zeros((1, 8), jnp.float32).at[0, :5].set(agg_weights)

    h = x
    for enc_W, enc_b, pre_W, pre_b, post_W, post_b, lin_W, lin_b, bn_g, bn_b in (
        (enc_W0, enc_b0, pre_W0, pre_b0, post_W0, post_b0, lin_W0, lin_b0, bn_g0, bn_b0),
        (enc_W1, enc_b1, pre_W1, pre_b1, post_W1, post_b1, lin_W1, lin_b1, bn_g1, bn_b1),
    ):
        h_p = jnp.pad(h, ((0, _NPC - n), (0, 0)))
        h = _conv_layer(h_p, src, dst, edge_attr, edge_emb, aggw8,
                        enc_W, enc_b, pre_W, pre_b, post_W, post_b, lin_W, lin_b)[:n]
        h = _bn_relu(h, bn_g, bn_b)
    return h


# speedup vs baseline: 1.6734x; 1.0864x over previous
import functools

import jax
import jax.numpy as jnp
from jax import lax
from jax.experimental import pallas as pl
from jax.experimental.pallas import tpu as pltpu
from jax.experimental.pallas import tpu_sc as plsc

_INTERPRET = False

_NR = 112
_NRANGES = 96
_NP = _NR * _NRANGES
_NPC = 10240
_DP = 768
_NC = 8
_DC = _DP // _NC
_SB = 2000
_GB = 16
_EPAD = 50176
_DUMMY = _NR << 14


def _matmul_block_kernel(x_ref, w_ref, o_ref):
    o_ref[...] = jnp.dot(x_ref[...], w_ref[...], preferred_element_type=jnp.float32)


def _matmul(x, w, bm=None):
    m, k = x.shape
    n = w.shape[1]
    if bm is None or m % bm != 0:
        bm = m
    return pl.pallas_call(
        _matmul_block_kernel,
        out_shape=jax.ShapeDtypeStruct((m, n), jnp.float32),
        grid=(m // bm,),
        in_specs=[
            pl.BlockSpec((bm, k), lambda i: (i, 0)),
            pl.BlockSpec((k, n), lambda i: (0, 0)),
        ],
        out_specs=pl.BlockSpec((bm, n), lambda i: (i, 0)),
        interpret=_INTERPRET,
    )(x, w)


def _b_kernel(x_ref, ws_ref, b_ref):
    b_ref[...] = jnp.dot(x_ref[...], ws_ref[...], preferred_element_type=jnp.float32)


def _compute_b(x, ws, bm=1024):
    n, d = x.shape
    dout = ws.shape[1]
    return pl.pallas_call(
        _b_kernel,
        out_shape=jax.ShapeDtypeStruct((_NP, dout), jnp.float32),
        grid=(n // bm,),
        in_specs=[
            pl.BlockSpec((bm, d), lambda i: (i, 0)),
            pl.BlockSpec((d, dout), lambda i: (0, 0)),
        ],
        out_specs=pl.BlockSpec((bm, dout), lambda i: (i, 0)),
        interpret=_INTERPRET,
    )(x, ws)


def _combine_kernel(x_ref, s_ref, mn_ref, mx_ref, ssq_ref, aw_ref,
                    pl_ref, m1_ref, m234_ref, bias_ref, o_ref):
    d = x_ref.shape[1]
    cnt = s_ref[...][:, d:d + 1]
    has = cnt > 0.0
    c1 = jnp.maximum(cnt, 1.0)
    su = s_ref[...][:, :d]
    mean_u = jnp.where(has, su / c1, 0.0)
    msq_u = jnp.where(has, ssq_ref[...][:, :d] / c1, 0.0)
    var = msq_u - mean_u * mean_u
    std = aw_ref[0, 4] * jnp.sqrt(jnp.maximum(var, 0.0) + 1e-5)
    mn = jnp.where(has, mn_ref[...][:, :d], 0.0)
    mx = jnp.where(has, mx_ref[...][:, :d], 0.0)
    xv = x_ref[...]
    acc = jnp.dot(xv, pl_ref[0], preferred_element_type=jnp.float32)
    acc += jnp.dot(aw_ref[0, 0] * su, pl_ref[1], preferred_element_type=jnp.float32)
    acc += jnp.dot(aw_ref[0, 1] * mean_u, pl_ref[2], preferred_element_type=jnp.float32)
    acc += jnp.dot(aw_ref[0, 2] * mn, pl_ref[3], preferred_element_type=jnp.float32)
    acc += jnp.dot(aw_ref[0, 3] * mx, pl_ref[4], preferred_element_type=jnp.float32)
    acc += jnp.dot(std, pl_ref[5], preferred_element_type=jnp.float32)
    acc += (aw_ref[0, 0] * cnt) * jnp.dot(xv, m1_ref[...], preferred_element_type=jnp.float32)
    acc += jnp.where(has, 1.0, 0.0) * jnp.dot(xv, m234_ref[...], preferred_element_type=jnp.float32)
    o_ref[...] = acc + bias_ref[...]


def _combine(x, s, mn, mx, ssq, aggw, plw, m1, m234, bias, bm=512):
    n, d = x.shape
    agg_spec = pl.BlockSpec((bm, _DP), lambda i: (i, 0))
    w_spec = pl.BlockSpec((d, d), lambda i: (0, 0))
    return pl.pallas_call(
        _combine_kernel,
        out_shape=jax.ShapeDtypeStruct((n, d), jnp.float32),
        grid=(n // bm,),
        in_specs=[
            pl.BlockSpec((bm, d), lambda i: (i, 0)),
            agg_spec, agg_spec, agg_spec, agg_spec,
            pl.BlockSpec((1, 8), lambda i: (0, 0)),
            pl.BlockSpec((6, d, d), lambda i: (0, 0, 0)),
            w_spec, w_spec,
            pl.BlockSpec((1, d), lambda i: (0, 0)),
        ],
        out_specs=pl.BlockSpec((bm, d), lambda i: (i, 0)),
        interpret=_INTERPRET,
    )(x, s, mn, mx, ssq, aggw, plw, m1, m234, bias)


def _colstats_kernel(h_ref, o_ref):
    @pl.when(pl.program_id(0) == 0)
    def _():
        o_ref[...] = jnp.zeros_like(o_ref)
    hv = h_ref[...]
    o_ref[0, :] += jnp.sum(hv, axis=0)
    o_ref[1, :] += jnp.sum(hv * hv, axis=0)


def _bn_relu_apply_kernel(h_ref, st_ref, g_ref, b_ref, o_ref):
    n_total = st_ref[2, 0]
    mu = st_ref[0, :] / n_total
    var = st_ref[1, :] / n_total - mu * mu
    inv = jax.lax.rsqrt(var + 1e-5)
    o_ref[...] = jnp.maximum(
        (h_ref[...] - mu[None, :]) * (inv * g_ref[0, :])[None, :] + b_ref[0, :][None, :],
        0.0)


def _bn_relu(h, g, b, bm=1000):
    n, d = h.shape
    stats = pl.pallas_call(
        _colstats_kernel,
        out_shape=jax.ShapeDtypeStruct((3, d), jnp.float32),
        grid=(n // bm,),
        in_specs=[pl.BlockSpec((bm, d), lambda i: (i, 0))],
        out_specs=pl.BlockSpec((3, d), lambda i: (0, 0)),
        interpret=_INTERPRET,
    )(h)
    stats = stats.at[2, 0].set(float(n))
    return pl.pallas_call(
        _bn_relu_apply_kernel,
        out_shape=jax.ShapeDtypeStruct((n, d), jnp.float32),
        grid=(n // bm,),
        in_specs=[
            pl.BlockSpec((bm, d), lambda i: (i, 0)),
            pl.BlockSpec((3, d), lambda i: (0, 0)),
            pl.BlockSpec((1, d), lambda i: (0, 0)),
            pl.BlockSpec((1, d), lambda i: (0, 0)),
        ],
        out_specs=pl.BlockSpec((bm, d), lambda i: (i, 0)),
        interpret=_INTERPRET,
    )(h, stats, g.reshape(1, d), b.reshape(1, d))


def _chunk_major_kernel(x_ref, o_ref):
    bm = x_ref.shape[0]
    o_ref[...] = x_ref[...].reshape(bm, _NC, _DC).transpose(1, 0, 2)


def _chunk_major(bp, bm=512):
    return pl.pallas_call(
        _chunk_major_kernel,
        out_shape=jax.ShapeDtypeStruct((_NC, _NP, _DC), jnp.float32),
        grid=(_NP // bm,),
        in_specs=[pl.BlockSpec((bm, _DP), lambda j: (j, 0))],
        out_specs=pl.BlockSpec((_NC, bm, _DC), lambda j: (0, j, 0)),
        interpret=_INTERPRET,
    )(bp)


def _sc_scatter_body(b2_hbm, t_hbm, dst_hbm, src_hbm, ea_hbm,
                     s_out, mn_out, mx_out, ssq_out,
                     t_loc, acc_s, acc_mn, acc_mx, acc_ssq,
                     e_dst, e_src, e_ea, o_packed, idx0, rb0):
    n_tiles = dst_hbm.shape[0] // _SB
    n_vecs = _SB // 16
    wid = lax.axis_index("s") * 2 + lax.axis_index("c")
    lane = lax.iota(jnp.int32, 16)

    @pl.loop(0, _EPAD // 16)
    def _(i):
        o_packed[pl.ds(i * 16, 16)] = jnp.full((16,), _DUMMY, jnp.int32)

    def scan_range(base):
        def tile_body(tp, cur):
            pltpu.sync_copy(dst_hbm.at[pl.ds(tp * _SB, _SB)], e_dst)
            pltpu.sync_copy(src_hbm.at[pl.ds(tp * _SB, _SB)], e_src)
            pltpu.sync_copy(ea_hbm.at[pl.ds(tp * _SB, _SB)], e_ea)

            def vec_body(j, cur):
                vd = e_dst[pl.ds(j * 16, 16)]
                mask = (vd >= base) & (vd < base + _NR)
                packed = (e_src[pl.ds(j * 16, 16)]
                          | ((vd - base) << 14)
                          | (e_ea[pl.ds(j * 16, 16)] << 22))
                pos = plsc.cumsum(mask.astype(jnp.int32)) + (cur - 1)
                plsc.store_scatter(o_packed, [pos], packed, mask=mask)
                pc = plsc.all_reduce_population_count(mask)[0]
                return cur + pc

            return lax.fori_loop(0, n_vecs, vec_body, cur)

        cur = lax.fori_loop(0, n_tiles, tile_body, jnp.int32(0))
        for i in range(_GB // 16):
            plsc.store_scatter(o_packed, [cur + i * 16 + lane],
                               jnp.full((16,), _DUMMY, jnp.int32))
        return cur

    def process_chunk(c, count, base):
        col = c * _DC
        pltpu.sync_copy(t_hbm.at[:, pl.ds(col, _DC)], t_loc)

        @pl.loop(0, _NR + 8)
        def _(i):
            for g in range(_DC // 16):
                sl = pl.ds(g * 16, 16)
                acc_s[i, sl] = jnp.zeros((16,), jnp.float32)
                acc_ssq[i, sl] = jnp.zeros((16,), jnp.float32)
                acc_mn[i, sl] = jnp.full((16,), jnp.inf, jnp.float32)
                acc_mx[i, sl] = jnp.full((16,), -jnp.inf, jnp.float32)

        n_batches = (count + _GB - 1) // _GB

        def batch_body(bi, _):
            b0 = bi * _GB
            for i in range(_GB // 16):
                wv = o_packed[pl.ds(b0 + i * 16, 16)]
                idx0[pl.ds(i * 16, 16)] = (wv & 16383) + c * _NP
            pltpu.sync_copy(b2_hbm.at[idx0], rb0)

            for i in range(_GB // 16):
                wv = o_packed[pl.ds(b0 + i * 16, 16)]
                for ln in range(16):
                    w = wv[ln]
                    dl = (w >> 14) & 255
                    eav = (w >> 22) & 31
                    k = i * 16 + ln
                    for g in range(_DC // 16):
                        sl = pl.ds(g * 16, 16)
                        u = rb0[k, sl] + t_loc[eav, sl]
                        plsc.addupdate(acc_s.at[dl, sl], u)
                        plsc.addupdate(acc_ssq.at[dl, sl], u * u)
                        acc_mn[dl, sl] = jnp.minimum(acc_mn[dl, sl], u)
                        acc_mx[dl, sl] = jnp.maximum(acc_mx[dl, sl], u)
            return 0

        lax.fori_loop(0, n_batches, batch_body, 0)

        pltpu.sync_copy(acc_s.at[pl.ds(0, _NR)],
                        s_out.at[pl.ds(base, _NR), pl.ds(col, _DC)])
        pltpu.sync_copy(acc_mn.at[pl.ds(0, _NR)],
                        mn_out.at[pl.ds(base, _NR), pl.ds(col, _DC)])
        pltpu.sync_copy(acc_mx.at[pl.ds(0, _NR)],
                        mx_out.at[pl.ds(base, _NR), pl.ds(col, _DC)])
        pltpu.sync_copy(acc_ssq.at[pl.ds(0, _NR)],
                        ssq_out.at[pl.ds(base, _NR), pl.ds(col, _DC)])

    def range_body(rr, _):
        base = (wid * 3 + rr) * _NR
        with jax.named_scope("edge_scan"):
            count = scan_range(base)

        def chunk_body(c, _):
            process_chunk(c, count, base)
            return 0

        with jax.named_scope("chunk_accum"):
            lax.fori_loop(0, _NC, chunk_body, 0)
        return 0

    lax.fori_loop(0, 3, range_body, 0)


def _sc_scatter(b_t, t_table, dst, src, ea):
    b2 = b_t.reshape(_NC * _NP, _DC)
    f32 = jnp.float32
    mesh = plsc.VectorSubcoreMesh(core_axis_name="c", subcore_axis_name="s")
    kern = pl.kernel(
        _sc_scatter_body,
        out_type=[jax.ShapeDtypeStruct((_NP, _DP), f32)] * 4,
        mesh=mesh,
        compiler_params=pltpu.CompilerParams(use_tc_tiling_on_sc=False, needs_layout_passes=False),
        scratch_types=[
            pltpu.VMEM((32, _DC), f32),
            pltpu.VMEM((_NR + 8, _DC), f32),
            pltpu.VMEM((_NR + 8, _DC), f32),
            pltpu.VMEM((_NR + 8, _DC), f32),
            pltpu.VMEM((_NR + 8, _DC), f32),
            pltpu.VMEM((_SB,), jnp.int32),
            pltpu.VMEM((_SB,), jnp.int32),
            pltpu.VMEM((_SB,), jnp.int32),
            pltpu.VMEM((_EPAD,), jnp.int32),
            pltpu.VMEM((_GB,), jnp.int32),
            pltpu.VMEM((_GB, _DC), f32),
        ],
    )
    return kern(b2, t_table, dst, src, ea)


def _ttable_kernel(emb_ref, encw_ref, encb_ref, we_ref, preb_ref, o_ref):
    enc = jnp.dot(emb_ref[...], encw_ref[...], preferred_element_type=jnp.float32)
    enc = enc + encb_ref[...]
    out = jnp.dot(enc, we_ref[...], preferred_element_type=jnp.float32) + preb_ref[...]
    mark = lax.broadcasted_iota(jnp.int32, out.shape, 1) == we_ref.shape[0]
    o_ref[...] = out + jnp.where(mark, 1.0, 0.0)


def _ttable(edge_emb, enc_W, enc_b, we_p, pre_b_p):
    d = enc_W.shape[1]
    emb32 = jnp.zeros((32, 16), jnp.float32).at[:20, :10].set(edge_emb)
    encw16 = jnp.zeros((16, d), jnp.float32).at[:10].set(enc_W)
    return pl.pallas_call(
        _ttable_kernel,
        out_shape=jax.ShapeDtypeStruct((32, _DP), jnp.float32),
        interpret=_INTERPRET,
    )(emb32, encw16, enc_b.reshape(1, d), we_p, pre_b_p.reshape(1, _DP))


def _conv_layer(x_p, src, dst, edge_attr, edge_emb, aggw8,
                enc_W, enc_b, pre_W, pre_b, post_W, post_b, lin_W, lin_b):
    d = x_p.shape[1]
    pad_c = ((0, 0), (0, _DP - d))
    ws = jnp.pad(pre_W[d:2 * d], pad_c)
    we = jnp.pad(pre_W[2 * d:], pad_c)
    pre_b_p = jnp.pad(pre_b, (0, _DP - d))
    b_p = _compute_b(x_p, ws)
    b_t = _chunk_major(b_p)
    t_table = _ttable(edge_emb, enc_W, enc_b, we, pre_b_p)
    s, mn, mx, ssq = _sc_scatter(b_t, t_table, dst, src, edge_attr)
    pw = jnp.concatenate([post_W, post_b[None, :]], axis=0)
    plw_full = _matmul(pw, lin_W, bm=None)
    bias = (plw_full[6 * d] + lin_b)[None, :]
    plw = plw_full[:6 * d].reshape(6, d, d)
    aw = aggw8[0]
    m1 = _matmul(pre_W[:d], plw[1], bm=None)
    m234 = _matmul(pre_W[:d],
                   aw[1] * plw[2] + aw[2] * plw[3] + aw[3] * plw[4], bm=None)
    return _combine(x_p, s, mn, mx, ssq, aggw8, plw, m1, m234, bias)


def kernel(x, edge_index, edge_attr, edge_emb, agg_weights,
           enc_W0, enc_b0, pre_W0, pre_b0, post_W0, post_b0, lin_W0, lin_b0, bn_g0, bn_b0,
           enc_W1, enc_b1, pre_W1, pre_b1, post_W1, post_b1, lin_W1, lin_b1, bn_g1, bn_b1):
    n, d = x.shape
    src = edge_index[0]
    dst = edge_index[1]
    aggw8 = jnp.zeros((1, 8), jnp.float32).at[0, :5].set(agg_weights)

    h = x
    for enc_W, enc_b, pre_W, pre_b, post_W, post_b, lin_W, lin_b, bn_g, bn_b in (
        (enc_W0, enc_b0, pre_W0, pre_b0, post_W0, post_b0, lin_W0, lin_b0, bn_g0, bn_b0),
        (enc_W1, enc_b1, pre_W1, pre_b1, post_W1, post_b1, lin_W1, lin_b1, bn_g1, bn_b1),
    ):
        h_p = jnp.pad(h, ((0, _NPC - n), (0, 0)))
        h = _conv_layer(h_p, src, dst, edge_attr, edge_emb, aggw8,
                        enc_W, enc_b, pre_W, pre_b, post_W, post_b, lin_W, lin_b)[:n]
        h = _bn_relu(h, bn_g, bn_b)
    return h
